# baseline (speedup 1.0000x reference)
import functools

import jax
import jax.numpy as jnp
import numpy as np
from jax import lax
from jax.experimental import pallas as pl
from jax.experimental.pallas import tpu as pltpu
from jax.experimental.pallas import tpu_sc as plsc

N = 50000
E = 800000
T = 400000
NUM_SPECIES = 7
N_RBF = 16
RC_R = 0.51
RMIN_R = 0.08
RC_A = 0.35
RMIN_A = 0.08
N_DIST = 8
N_ANG = 4
ETA_R = 19.7
ETA_A = 12.5
ZETA = 14.1
N_PAIRS = NUM_SPECIES * (NUM_SPECIES + 1) // 2
ANG_SUB = N_DIST * N_ANG

EB = 32000
TB = 16000

L = 16
NC = 2
NS = 16

SENTINEL = 1 << 20

A_PASSES = 14
A_CHUNK = 50000
A_ROWS = NC * A_PASSES * A_CHUNK
A_ACC = A_CHUNK + L
T_PAD = 409600
A_IPW = T_PAD // NS
A_SCH = 3200
A_K = 256

R_PASSES = 2
R_CHUNK = 87552
R_ROWS = NC * R_PASSES * R_CHUNK
R_ACC = R_CHUNK + L
E_PAD = 802816
R_IPW = E_PAD // NS
R_SCH = 3584
R_K = 768


def _iota_row(n):
    return jax.lax.broadcasted_iota(jnp.int32, (1, n), 1).astype(jnp.float32)


def _iota_col(n):
    return jax.lax.broadcasted_iota(jnp.int32, (n, 1), 0).astype(jnp.float32)


def _radial_body(d_ref, out_ref):
    d = d_ref[0]
    fc = jnp.where(d <= RC_R,
                   0.5 * (jnp.cos(jnp.pi * jnp.minimum(d, RC_R) / RC_R) + 1.0),
                   0.0)
    shf = RMIN_R + _iota_col(N_RBF) * ((RC_R - RMIN_R) / N_RBF)
    out_ref[...] = 0.25 * jnp.exp(-ETA_R * (d - shf) ** 2) * fc


def _angular_body(r_ref, out_ref):
    r = r_ref[0]
    a = r[0:3, :]
    b = r[3:6, :]
    d1 = jnp.sqrt(jnp.sum(a * a, axis=0, keepdims=True))
    d2 = jnp.sqrt(jnp.sum(b * b, axis=0, keepdims=True))
    dot = jnp.sum(a * b, axis=0, keepdims=True)
    cos_t = 0.95 * dot / (d1 * d2 + 1e-10)
    c = jnp.clip(cos_t, -0.99, 0.99)
    s = jnp.sqrt(1.0 - c * c)
    shf_z = _iota_col(N_ANG) * (np.pi / N_ANG) + (np.pi / (2 * N_ANG))
    base = 0.5 * (1.0 + c * jnp.cos(shf_z) + s * jnp.sin(shf_z))
    factor1 = base ** ZETA
    dmean = 0.5 * (d1 + d2)
    shf_a = RMIN_A + _iota_col(N_DIST) * ((RC_A - RMIN_A) / N_DIST)
    factor2 = jnp.exp(-ETA_A * (dmean - shf_a) ** 2)

    def fcut(d):
        return jnp.where(d <= RC_A,
                         0.5 * (jnp.cos(jnp.pi * jnp.minimum(d, RC_A) / RC_A) + 1.0),
                         0.0)

    fcj12 = fcut(d1) * fcut(d2)
    scale = (2.0 ** (1.0 - ZETA)) * fcj12
    k32 = jax.lax.broadcasted_iota(jnp.int32, (ANG_SUB, 1), 0)
    s8 = (jax.lax.broadcasted_iota(jnp.int32, (ANG_SUB, N_DIST), 1)
          == k32 // N_ANG).astype(jnp.float32)
    s4 = (jax.lax.broadcasted_iota(jnp.int32, (ANG_SUB, N_ANG), 1)
          == k32 % N_ANG).astype(jnp.float32)
    f2w = jax.lax.dot(s8, factor2, preferred_element_type=jnp.float32)
    f1w = jax.lax.dot(s4, factor1, preferred_element_type=jnp.float32)
    out_ref[...] = scale * f2w * f1w


NB = 1000


def _assemble_body(rad_ref, ang_ref, out_ref):
    out_ref[:, 0:NUM_SPECIES * N_RBF] = rad_ref[...]
    out_ref[:, NUM_SPECIES * N_RBF:] = ang_ref[...]


def _assemble(rad2, ang2):
    width = NUM_SPECIES * N_RBF + N_PAIRS * ANG_SUB
    return pl.pallas_call(
        _assemble_body,
        grid=(N // NB,),
        in_specs=[pl.BlockSpec((NB, NUM_SPECIES * N_RBF), lambda i: (i, 0)),
                  pl.BlockSpec((NB, N_PAIRS * ANG_SUB), lambda i: (i, 0))],
        out_specs=pl.BlockSpec((NB, width), lambda i: (i, 0)),
        out_shape=jax.ShapeDtypeStruct((N, width), jnp.float32),
    )(rad2, ang2)


def _features(d_ij, r6):
    d3 = d_ij.reshape(E // EB, 1, EB)
    rfv_t = pl.pallas_call(
        _radial_body,
        grid=(E // EB,),
        in_specs=[pl.BlockSpec((1, 1, EB), lambda i: (i, 0, 0))],
        out_specs=pl.BlockSpec((N_RBF, EB), lambda i: (0, i)),
        out_shape=jax.ShapeDtypeStruct((N_RBF, E), jnp.float32),
    )(d3)
    r6t3 = r6.reshape(T // TB, TB, 6).transpose(0, 2, 1)
    ang_t = pl.pallas_call(
        _angular_body,
        grid=(T // TB,),
        in_specs=[pl.BlockSpec((1, 6, TB), lambda i: (i, 0, 0))],
        out_specs=pl.BlockSpec((ANG_SUB, TB), lambda i: (0, i)),
        out_shape=jax.ShapeDtypeStruct((ANG_SUB, T), jnp.float32),
    )(r6t3)
    return rfv_t.T, ang_t.T


def _iota16():
    return jax.lax.broadcasted_iota(jnp.int32, (L,), 0)


def _preset(buf, n, value):
    vec = jnp.full((L,), value, jnp.int32)

    @pl.loop(0, n // L)
    def _(i):
        buf[pl.ds(i * L, L)] = vec


def _zero_rows(zbuf, acc, row0, nrows, zrows):
    nfull = nrows // zrows
    rem = nrows - nfull * zrows

    @pl.loop(0, nfull)
    def _(i):
        pltpu.sync_copy(zbuf, acc.at[pl.ds(row0 + i * zrows, zrows)])

    if rem:
        pltpu.sync_copy(zbuf.at[pl.ds(0, rem)],
                        acc.at[pl.ds(row0 + nfull * zrows, rem)])


def _flush(rows_hbm, acc, ids_buf, dest_buf, rows_buf, cnt_ref, trash):
    pltpu.sync_copy(rows_hbm.at[ids_buf], rows_buf)
    pltpu.sync_copy(rows_buf, acc.at[dest_buf], add=True)
    _preset(dest_buf, dest_buf.shape[0], trash)
    cnt_ref[0] = 0


def _scan_block(dbuf, nitems, id_base, lo, chunk, trash, k,
                ids_buf, dest_buf, rows_buf, cnt_ref, rows_hbm, acc):
    @pl.loop(0, nitems // L)
    def _(c16):
        d16 = dbuf[pl.ds(c16 * L, L)]
        rel = d16 - lo
        m = (rel >= 0) & (rel < chunk)
        cnt = cnt_ref[0]
        plsc.store_compressed(dest_buf.at[pl.ds(cnt, L)], rel, mask=m)
        ids16 = (id_base + c16 * L) + _iota16()
        plsc.store_compressed(ids_buf.at[pl.ds(cnt, L)], ids16, mask=m)
        cnt_ref[0] = cnt + jnp.sum(m.astype(jnp.int32))

        @pl.when(cnt_ref[0] > k - L)
        def _():
            _flush(rows_hbm, acc, ids_buf, dest_buf, rows_buf, cnt_ref, trash)


def _angular_sc(ang, central_pad, s1_pad, s2_pad, zeros):
    mesh = plsc.VectorSubcoreMesh(core_axis_name="c", subcore_axis_name="s")

    @functools.partial(
        pl.kernel,
        out_type=(jax.ShapeDtypeStruct((A_ROWS, ANG_SUB), jnp.float32),
                  jax.ShapeDtypeStruct((T_PAD,), jnp.int32)),
        mesh=mesh,
        compiler_params=pltpu.CompilerParams(needs_layout_passes=False,
                                             use_tc_tiling_on_sc=False),
        scratch_types=[
            pltpu.VMEM((A_SCH,), jnp.int32),
            pltpu.VMEM((A_SCH,), jnp.int32),
            pltpu.VMEM((A_SCH,), jnp.int32),
            pltpu.VMEM((A_SCH,), jnp.int32),
            pltpu.VMEM((A_K,), jnp.int32),
            pltpu.VMEM((A_K,), jnp.int32),
            pltpu.VMEM((A_K, ANG_SUB), jnp.float32),
            pltpu.VMEM((192, ANG_SUB), jnp.float32),
            pltpu.SMEM((1,), jnp.int32),
            pltpu.VMEM_SHARED((A_ACC, ANG_SUB), jnp.float32),
        ],
    )
    def k(ang_hbm, c_hbm, s1_hbm, s2_hbm, z_hbm, out_hbm, dest_hbm,
          cbuf, s1buf, s2buf, dbuf, ids_buf, dest_buf, rows_buf, zbuf,
          cnt_ref, acc):
        cid = lax.axis_index("c")
        sid = lax.axis_index("s")
        base = sid * A_IPW

        pltpu.sync_copy(z_hbm, zbuf)
        _preset(dest_buf, A_K, A_CHUNK)
        _preset(ids_buf, A_K, 0)
        cnt_ref[0] = 0

        @pl.loop(0, A_IPW // A_SCH)
        def _(blk):
            off = base + blk * A_SCH
            pltpu.sync_copy(c_hbm.at[pl.ds(off, A_SCH)], cbuf)
            pltpu.sync_copy(s1_hbm.at[pl.ds(off, A_SCH)], s1buf)
            pltpu.sync_copy(s2_hbm.at[pl.ds(off, A_SCH)], s2buf)

            @pl.loop(0, A_SCH // L)
            def _(c16):
                slc = pl.ds(c16 * L, L)
                s1 = s1buf[slc]
                s2 = s2buf[slc]
                mn = jnp.minimum(s1, s2)
                mx = jnp.maximum(s1, s2)
                pr = mn * NUM_SPECIES - ((mn * (mn - 1)) >> 1) + (mx - mn)
                dbuf[slc] = cbuf[slc] * N_PAIRS + pr

            pltpu.sync_copy(dbuf, dest_hbm.at[pl.ds(off, A_SCH)])

        plsc.subcore_barrier()

        for p in range(A_PASSES):
            q = cid * A_PASSES + p
            lo = q * A_CHUNK

            _zero_rows(zbuf, acc, sid * (A_ACC // NS), A_ACC // NS, 192)
            plsc.subcore_barrier()

            @pl.loop(0, A_IPW // A_SCH)
            def _(blk):
                off = base + blk * A_SCH
                pltpu.sync_copy(dest_hbm.at[pl.ds(off, A_SCH)], dbuf)
                _scan_block(dbuf, A_SCH, off, lo, A_CHUNK, A_CHUNK, A_K,
                            ids_buf, dest_buf, rows_buf, cnt_ref, ang_hbm, acc)

            _flush(ang_hbm, acc, ids_buf, dest_buf, rows_buf, cnt_ref, A_CHUNK)
            plsc.subcore_barrier()

            wrows = A_CHUNK // NS
            pltpu.sync_copy(acc.at[pl.ds(sid * wrows, wrows)],
                            out_hbm.at[pl.ds(lo + sid * wrows, wrows)])
            plsc.subcore_barrier()

    return k(ang, central_pad, s1_pad, s2_pad, zeros)


def _radial_sc(rfv, i0_pad, i1_pad, atom_index, zeros):
    mesh = plsc.VectorSubcoreMesh(core_axis_name="c", subcore_axis_name="s")

    @functools.partial(
        pl.kernel,
        out_type=(jax.ShapeDtypeStruct((R_ROWS, N_RBF), jnp.float32),
                  jax.ShapeDtypeStruct((E_PAD,), jnp.int32),
                  jax.ShapeDtypeStruct((E_PAD,), jnp.int32)),
        mesh=mesh,
        compiler_params=pltpu.CompilerParams(needs_layout_passes=False,
                                             use_tc_tiling_on_sc=False),
        scratch_types=[
            pltpu.VMEM((N // 4,), jnp.int32),
            pltpu.VMEM((R_SCH,), jnp.int32),
            pltpu.VMEM((R_SCH,), jnp.int32),
            pltpu.VMEM((R_K,), jnp.int32),
            pltpu.VMEM((R_K,), jnp.int32),
            pltpu.VMEM((R_K, N_RBF), jnp.float32),
            pltpu.VMEM((256, N_RBF), jnp.float32),
            pltpu.SMEM((1,), jnp.int32),
            pltpu.VMEM_SHARED((R_ACC, N_RBF), jnp.float32),
        ],
    )
    def k(rfv_hbm, i0_hbm, i1_hbm, at_hbm, z_hbm, out_hbm, dA_hbm, dB_hbm,
          atv, buf0, buf1, ids_buf, dest_buf, rows_buf, zbuf,
          cnt_ref, acc):
        cid = lax.axis_index("c")
        sid = lax.axis_index("s")
        base = sid * R_IPW

        pltpu.sync_copy(z_hbm, zbuf)
        _preset(dest_buf, R_K, R_CHUNK)
        _preset(ids_buf, R_K, 0)
        cnt_ref[0] = 0
        pltpu.sync_copy(at_hbm, atv)

        @pl.loop(0, R_IPW // R_SCH)
        def _(blk):
            off = base + blk * R_SCH
            pltpu.sync_copy(i0_hbm.at[pl.ds(off, R_SCH)], buf0)
            pltpu.sync_copy(i1_hbm.at[pl.ds(off, R_SCH)], buf1)

            @pl.loop(0, R_SCH // L)
            def _(c16):
                slc = pl.ds(c16 * L, L)
                i0 = buf0[slc]
                i1 = buf1[slc]

                def species(i):
                    ic = jnp.minimum(i, N - 1)
                    w = plsc.load_gather(atv, [ic >> 2])
                    return lax.shift_right_logical(w, (ic & 3) * 8) & 7

                s0 = species(i0)
                s1 = species(i1)
                buf0[slc] = i0 * NUM_SPECIES + s1
                buf1[slc] = i1 * NUM_SPECIES + s0

            pltpu.sync_copy(buf0, dA_hbm.at[pl.ds(off, R_SCH)])
            pltpu.sync_copy(buf1, dB_hbm.at[pl.ds(off, R_SCH)])

        plsc.subcore_barrier()

        for p in range(R_PASSES):
            q = cid * R_PASSES + p
            lo = q * R_CHUNK

            _zero_rows(zbuf, acc, sid * (R_ACC // NS), R_ACC // NS, 256)
            plsc.subcore_barrier()

            for d_hbm in (dA_hbm, dB_hbm):
                @pl.loop(0, R_IPW // R_SCH)
                def _(blk, d_hbm=d_hbm):
                    off = base + blk * R_SCH
                    pltpu.sync_copy(d_hbm.at[pl.ds(off, R_SCH)], buf0)
                    _scan_block(buf0, R_SCH, off, lo, R_CHUNK, R_CHUNK, R_K,
                                ids_buf, dest_buf, rows_buf, cnt_ref,
                                rfv_hbm, acc)

            _flush(rfv_hbm, acc, ids_buf, dest_buf, rows_buf, cnt_ref, R_CHUNK)
            plsc.subcore_barrier()

            wrows = R_CHUNK // NS
            pltpu.sync_copy(acc.at[pl.ds(sid * wrows, wrows)],
                            out_hbm.at[pl.ds(lo + sid * wrows, wrows)])
            plsc.subcore_barrier()

    return k(rfv, i0_pad, i1_pad, atom_index, zeros)


def _pad_i32(x, n):
    return jnp.pad(x.astype(jnp.int32), (0, n - x.shape[0]),
                   constant_values=SENTINEL)


def kernel(d_ij, pair_indices, atom_index, central_atom_index,
           angular_species12, angular_r_ij):
    r6 = angular_r_ij.reshape(T, 6)
    rfv, ang = _features(d_ij, r6)

    i0 = _pad_i32(pair_indices[0], E_PAD)
    i1 = _pad_i32(pair_indices[1], E_PAD)
    ai = atom_index.astype(jnp.int32).reshape(N // 4, 4)
    ai_packed = (ai[:, 0] | (ai[:, 1] << 8) | (ai[:, 2] << 16)
                 | (ai[:, 3] << 24))
    rzeros = jnp.zeros((256, N_RBF), jnp.float32)
    rad_out, _, _ = _radial_sc(rfv, i0, i1, ai_packed, rzeros)
    radial_aev = rad_out[:N * NUM_SPECIES].reshape(N, NUM_SPECIES * N_RBF)

    central = _pad_i32(central_atom_index, T_PAD)
    s1 = jnp.pad(angular_species12[0].astype(jnp.int32), (0, T_PAD - T))
    s2 = jnp.pad(angular_species12[1].astype(jnp.int32), (0, T_PAD - T))
    azeros = jnp.zeros((192, ANG_SUB), jnp.float32)
    ang_out, _ = _angular_sc(ang, central, s1, s2, azeros)
    angular_aev = ang_out.reshape(N, N_PAIRS * ANG_SUB)

    return _assemble(radial_aev, angular_aev)


# speedup vs baseline: 1.0141x; 1.0141x over previous
import functools

import jax
import jax.numpy as jnp
import numpy as np
from jax import lax
from jax.experimental import pallas as pl
from jax.experimental.pallas import tpu as pltpu
from jax.experimental.pallas import tpu_sc as plsc

N = 50000
E = 800000
T = 400000
NUM_SPECIES = 7
N_RBF = 16
RC_R = 0.51
RMIN_R = 0.08
RC_A = 0.35
RMIN_A = 0.08
N_DIST = 8
N_ANG = 4
ETA_R = 19.7
ETA_A = 12.5
ZETA = 14.1
N_PAIRS = NUM_SPECIES * (NUM_SPECIES + 1) // 2
ANG_SUB = N_DIST * N_ANG

EB = 32000
TB = 16000

L = 16
NC = 2
NS = 16

SENTINEL = 1 << 20

A_PASSES = 14
A_CHUNK = 50000
A_ROWS = NC * A_PASSES * A_CHUNK
A_ACC = A_CHUNK + L
T_PAD = 409600
A_IPW = T_PAD // NS
A_SCH = 3200
A_K = 256

R_PASSES = 2
R_CHUNK = 87552
R_ROWS = NC * R_PASSES * R_CHUNK
R_ACC = R_CHUNK + L
E_PAD = 802816
R_IPW = E_PAD // NS
R_SCH = 3584
R_K = 512


def _iota_row(n):
    return jax.lax.broadcasted_iota(jnp.int32, (1, n), 1).astype(jnp.float32)


def _iota_col(n):
    return jax.lax.broadcasted_iota(jnp.int32, (n, 1), 0).astype(jnp.float32)


def _radial_body(d_ref, out_ref):
    d = d_ref[0]
    fc = jnp.where(d <= RC_R,
                   0.5 * (jnp.cos(jnp.pi * jnp.minimum(d, RC_R) / RC_R) + 1.0),
                   0.0)
    shf = RMIN_R + _iota_col(N_RBF) * ((RC_R - RMIN_R) / N_RBF)
    out_ref[...] = 0.25 * jnp.exp(-ETA_R * (d - shf) ** 2) * fc


def _angular_body(r_ref, out_ref):
    r = r_ref[0]
    a = r[0:3, :]
    b = r[3:6, :]
    d1 = jnp.sqrt(jnp.sum(a * a, axis=0, keepdims=True))
    d2 = jnp.sqrt(jnp.sum(b * b, axis=0, keepdims=True))
    dot = jnp.sum(a * b, axis=0, keepdims=True)
    cos_t = 0.95 * dot / (d1 * d2 + 1e-10)
    c = jnp.clip(cos_t, -0.99, 0.99)
    s = jnp.sqrt(1.0 - c * c)
    shf_z = _iota_col(N_ANG) * (np.pi / N_ANG) + (np.pi / (2 * N_ANG))
    base = 0.5 * (1.0 + c * jnp.cos(shf_z) + s * jnp.sin(shf_z))
    factor1 = base ** ZETA
    dmean = 0.5 * (d1 + d2)
    shf_a = RMIN_A + _iota_col(N_DIST) * ((RC_A - RMIN_A) / N_DIST)
    factor2 = jnp.exp(-ETA_A * (dmean - shf_a) ** 2)

    def fcut(d):
        return jnp.where(d <= RC_A,
                         0.5 * (jnp.cos(jnp.pi * jnp.minimum(d, RC_A) / RC_A) + 1.0),
                         0.0)

    fcj12 = fcut(d1) * fcut(d2)
    scale = (2.0 ** (1.0 - ZETA)) * fcj12
    k32 = jax.lax.broadcasted_iota(jnp.int32, (ANG_SUB, 1), 0)
    s8 = (jax.lax.broadcasted_iota(jnp.int32, (ANG_SUB, N_DIST), 1)
          == k32 // N_ANG).astype(jnp.float32)
    s4 = (jax.lax.broadcasted_iota(jnp.int32, (ANG_SUB, N_ANG), 1)
          == k32 % N_ANG).astype(jnp.float32)
    f2w = jax.lax.dot(s8, factor2, preferred_element_type=jnp.float32)
    f1w = jax.lax.dot(s4, factor1, preferred_element_type=jnp.float32)
    out_ref[...] = scale * f2w * f1w


NB = 1000


def _assemble_body(rad_ref, ang_ref, out_ref):
    out_ref[:, 0:NUM_SPECIES * N_RBF] = rad_ref[...]
    out_ref[:, NUM_SPECIES * N_RBF:] = ang_ref[...]


def _assemble(rad2, ang2):
    width = NUM_SPECIES * N_RBF + N_PAIRS * ANG_SUB
    return pl.pallas_call(
        _assemble_body,
        grid=(N // NB,),
        in_specs=[pl.BlockSpec((NB, NUM_SPECIES * N_RBF), lambda i: (i, 0)),
                  pl.BlockSpec((NB, N_PAIRS * ANG_SUB), lambda i: (i, 0))],
        out_specs=pl.BlockSpec((NB, width), lambda i: (i, 0)),
        out_shape=jax.ShapeDtypeStruct((N, width), jnp.float32),
    )(rad2, ang2)


def _features(d_ij, r6):
    d3 = d_ij.reshape(E // EB, 1, EB)
    rfv_t = pl.pallas_call(
        _radial_body,
        grid=(E // EB,),
        in_specs=[pl.BlockSpec((1, 1, EB), lambda i: (i, 0, 0))],
        out_specs=pl.BlockSpec((N_RBF, EB), lambda i: (0, i)),
        out_shape=jax.ShapeDtypeStruct((N_RBF, E), jnp.float32),
    )(d3)
    r6t3 = r6.reshape(T // TB, TB, 6).transpose(0, 2, 1)
    ang_t = pl.pallas_call(
        _angular_body,
        grid=(T // TB,),
        in_specs=[pl.BlockSpec((1, 6, TB), lambda i: (i, 0, 0))],
        out_specs=pl.BlockSpec((ANG_SUB, TB), lambda i: (0, i)),
        out_shape=jax.ShapeDtypeStruct((ANG_SUB, T), jnp.float32),
    )(r6t3)
    return rfv_t.T, ang_t.T


def _iota16():
    return jax.lax.broadcasted_iota(jnp.int32, (L,), 0)


def _preset(buf, n, value):
    vec = jnp.full((L,), value, jnp.int32)

    @pl.loop(0, n // L)
    def _(i):
        buf[pl.ds(i * L, L)] = vec


def _zero_rows(zbuf, acc, row0, nrows, zrows):
    nfull = nrows // zrows
    rem = nrows - nfull * zrows

    @pl.loop(0, nfull)
    def _(i):
        pltpu.sync_copy(zbuf, acc.at[pl.ds(row0 + i * zrows, zrows)])

    if rem:
        pltpu.sync_copy(zbuf.at[pl.ds(0, rem)],
                        acc.at[pl.ds(row0 + nfull * zrows, rem)])


def _flush(rows_hbm, acc, ids_buf, dest_buf, rows_buf, cnt_ref, trash):
    pltpu.sync_copy(rows_hbm.at[ids_buf], rows_buf)
    pltpu.sync_copy(rows_buf, acc.at[dest_buf], add=True)
    _preset(dest_buf, dest_buf.shape[0], trash)
    cnt_ref[0] = 0


def _scan_block(dbuf, nitems, id_base, lo, chunk, trash, k,
                ids_buf, dest_buf, rows_buf, cnt_ref, rows_hbm, acc):
    @pl.loop(0, nitems // L)
    def _(c16):
        d16 = dbuf[pl.ds(c16 * L, L)]
        rel = d16 - lo
        m = (rel >= 0) & (rel < chunk)
        cnt = cnt_ref[0]
        plsc.store_compressed(dest_buf.at[pl.ds(cnt, L)], rel, mask=m)
        ids16 = (id_base + c16 * L) + _iota16()
        plsc.store_compressed(ids_buf.at[pl.ds(cnt, L)], ids16, mask=m)
        cnt_ref[0] = cnt + jnp.sum(m.astype(jnp.int32))

        @pl.when(cnt_ref[0] > k - L)
        def _():
            _flush(rows_hbm, acc, ids_buf, dest_buf, rows_buf, cnt_ref, trash)


def _angular_sc(ang, central_pad, s1_pad, s2_pad, zeros):
    mesh = plsc.VectorSubcoreMesh(core_axis_name="c", subcore_axis_name="s")

    @functools.partial(
        pl.kernel,
        out_type=(jax.ShapeDtypeStruct((A_ROWS, ANG_SUB), jnp.float32),
                  jax.ShapeDtypeStruct((T_PAD,), jnp.int32)),
        mesh=mesh,
        compiler_params=pltpu.CompilerParams(needs_layout_passes=False,
                                             use_tc_tiling_on_sc=False),
        scratch_types=[
            pltpu.VMEM((A_SCH,), jnp.int32),
            pltpu.VMEM((A_SCH,), jnp.int32),
            pltpu.VMEM((A_SCH,), jnp.int32),
            pltpu.VMEM((A_SCH,), jnp.int32),
            pltpu.VMEM((A_K,), jnp.int32),
            pltpu.VMEM((A_K,), jnp.int32),
            pltpu.VMEM((A_K, ANG_SUB), jnp.float32),
            pltpu.VMEM((128, ANG_SUB), jnp.float32),
            pltpu.SMEM((1,), jnp.int32),
            pltpu.VMEM_SHARED((A_ACC, ANG_SUB), jnp.float32),
        ],
    )
    def k(ang_hbm, c_hbm, s1_hbm, s2_hbm, z_hbm, out_hbm, dest_hbm,
          cbuf, s1buf, s2buf, dbuf, ids_buf, dest_buf, rows_buf, zbuf,
          cnt_ref, acc):
        cid = lax.axis_index("c")
        sid = lax.axis_index("s")
        base = sid * A_IPW

        pltpu.sync_copy(z_hbm, zbuf)
        _preset(dest_buf, A_K, A_CHUNK)
        _preset(ids_buf, A_K, 0)
        cnt_ref[0] = 0

        @pl.loop(0, A_IPW // A_SCH)
        def _(blk):
            off = base + blk * A_SCH
            pltpu.sync_copy(c_hbm.at[pl.ds(off, A_SCH)], cbuf)
            pltpu.sync_copy(s1_hbm.at[pl.ds(off, A_SCH)], s1buf)
            pltpu.sync_copy(s2_hbm.at[pl.ds(off, A_SCH)], s2buf)

            @pl.loop(0, A_SCH // L)
            def _(c16):
                slc = pl.ds(c16 * L, L)
                s1 = s1buf[slc]
                s2 = s2buf[slc]
                mn = jnp.minimum(s1, s2)
                mx = jnp.maximum(s1, s2)
                pr = mn * NUM_SPECIES - ((mn * (mn - 1)) >> 1) + (mx - mn)
                dbuf[slc] = cbuf[slc] * N_PAIRS + pr

            pltpu.sync_copy(dbuf, dest_hbm.at[pl.ds(off, A_SCH)])

        plsc.subcore_barrier()

        for p in range(A_PASSES):
            q = cid * A_PASSES + p
            lo = q * A_CHUNK

            _zero_rows(zbuf, acc, sid * (A_ACC // NS), A_ACC // NS, 128)
            plsc.subcore_barrier()

            @pl.loop(0, A_IPW // A_SCH)
            def _(blk):
                off = base + blk * A_SCH
                pltpu.sync_copy(dest_hbm.at[pl.ds(off, A_SCH)], dbuf)
                _scan_block(dbuf, A_SCH, off, lo, A_CHUNK, A_CHUNK, A_K,
                            ids_buf, dest_buf, rows_buf, cnt_ref, ang_hbm, acc)

            _flush(ang_hbm, acc, ids_buf, dest_buf, rows_buf, cnt_ref, A_CHUNK)
            plsc.subcore_barrier()

            wrows = A_CHUNK // NS
            pltpu.sync_copy(acc.at[pl.ds(sid * wrows, wrows)],
                            out_hbm.at[pl.ds(lo + sid * wrows, wrows)])
            plsc.subcore_barrier()

    return k(ang, central_pad, s1_pad, s2_pad, zeros)


def _radial_sc(rfv, i0_pad, i1_pad, atom_index, zeros):
    mesh = plsc.VectorSubcoreMesh(core_axis_name="c", subcore_axis_name="s")

    @functools.partial(
        pl.kernel,
        out_type=(jax.ShapeDtypeStruct((R_ROWS, N_RBF), jnp.float32),
                  jax.ShapeDtypeStruct((E_PAD,), jnp.int32),
                  jax.ShapeDtypeStruct((E_PAD,), jnp.int32)),
        mesh=mesh,
        compiler_params=pltpu.CompilerParams(needs_layout_passes=False,
                                             use_tc_tiling_on_sc=False),
        scratch_types=[
            pltpu.VMEM((N // 4,), jnp.int32),
            pltpu.VMEM((R_SCH,), jnp.int32),
            pltpu.VMEM((R_SCH,), jnp.int32),
            pltpu.VMEM((R_SCH,), jnp.int32),
            pltpu.VMEM((R_K,), jnp.int32),
            pltpu.VMEM((R_K,), jnp.int32),
            pltpu.VMEM((R_K, N_RBF), jnp.float32),
            pltpu.VMEM((256, N_RBF), jnp.float32),
            pltpu.SMEM((1,), jnp.int32),
            pltpu.VMEM_SHARED((R_ACC, N_RBF), jnp.float32),
        ],
    )
    def k(rfv_hbm, i0_hbm, i1_hbm, at_hbm, z_hbm, out_hbm, dA_hbm, dB_hbm,
          atv, buf0, buf1, dbuf, ids_buf, dest_buf, rows_buf, zbuf,
          cnt_ref, acc):
        cid = lax.axis_index("c")
        sid = lax.axis_index("s")
        base = sid * R_IPW

        pltpu.sync_copy(z_hbm, zbuf)
        _preset(dest_buf, R_K, R_CHUNK)
        _preset(ids_buf, R_K, 0)
        cnt_ref[0] = 0
        pltpu.sync_copy(at_hbm, atv)

        @pl.loop(0, R_IPW // R_SCH)
        def _(blk):
            off = base + blk * R_SCH
            pltpu.sync_copy(i0_hbm.at[pl.ds(off, R_SCH)], buf0)
            pltpu.sync_copy(i1_hbm.at[pl.ds(off, R_SCH)], buf1)

            @pl.loop(0, R_SCH // L)
            def _(c16):
                slc = pl.ds(c16 * L, L)
                i0 = buf0[slc]
                i1 = buf1[slc]

                def species(i):
                    ic = jnp.minimum(i, N - 1)
                    w = plsc.load_gather(atv, [ic >> 2])
                    return lax.shift_right_logical(w, (ic & 3) * 8) & 7

                s0 = species(i0)
                s1 = species(i1)
                dbuf[slc] = i0 * NUM_SPECIES + s1
                buf0[slc] = i1 * NUM_SPECIES + s0

            pltpu.sync_copy(dbuf, dA_hbm.at[pl.ds(off, R_SCH)])
            pltpu.sync_copy(buf0, dB_hbm.at[pl.ds(off, R_SCH)])

        plsc.subcore_barrier()

        for p in range(R_PASSES):
            q = cid * R_PASSES + p
            lo = q * R_CHUNK

            _zero_rows(zbuf, acc, sid * (R_ACC // NS), R_ACC // NS, 256)
            plsc.subcore_barrier()

            for d_hbm in (dA_hbm, dB_hbm):
                @pl.loop(0, R_IPW // R_SCH)
                def _(blk, d_hbm=d_hbm):
                    off = base + blk * R_SCH
                    pltpu.sync_copy(d_hbm.at[pl.ds(off, R_SCH)], dbuf)
                    _scan_block(dbuf, R_SCH, off, lo, R_CHUNK, R_CHUNK, R_K,
                                ids_buf, dest_buf, rows_buf, cnt_ref,
                                rfv_hbm, acc)

            _flush(rfv_hbm, acc, ids_buf, dest_buf, rows_buf, cnt_ref, R_CHUNK)
            plsc.subcore_barrier()

            wrows = R_CHUNK // NS
            pltpu.sync_copy(acc.at[pl.ds(sid * wrows, wrows)],
                            out_hbm.at[pl.ds(lo + sid * wrows, wrows)])
            plsc.subcore_barrier()

    return k(rfv, i0_pad, i1_pad, atom_index, zeros)


def _pad_i32(x, n):
    return jnp.pad(x.astype(jnp.int32), (0, n - x.shape[0]),
                   constant_values=SENTINEL)


def kernel(d_ij, pair_indices, atom_index, central_atom_index,
           angular_species12, angular_r_ij):
    r6 = angular_r_ij.reshape(T, 6)
    rfv, ang = _features(d_ij, r6)

    i0 = _pad_i32(pair_indices[0], E_PAD)
    i1 = _pad_i32(pair_indices[1], E_PAD)
    ai = atom_index.astype(jnp.int32).reshape(N // 4, 4)
    ai_packed = (ai[:, 0] | (ai[:, 1] << 8) | (ai[:, 2] << 16)
                 | (ai[:, 3] << 24))
    rzeros = jnp.zeros((256, N_RBF), jnp.float32)
    rad_out, _, _ = _radial_sc(rfv, i0, i1, ai_packed, rzeros)
    radial_aev = rad_out[:N * NUM_SPECIES].reshape(N, NUM_SPECIES * N_RBF)

    central = _pad_i32(central_atom_index, T_PAD)
    s1 = jnp.pad(angular_species12[0].astype(jnp.int32), (0, T_PAD - T))
    s2 = jnp.pad(angular_species12[1].astype(jnp.int32), (0, T_PAD - T))
    azeros = jnp.zeros((128, ANG_SUB), jnp.float32)
    ang_out, _ = _angular_sc(ang, central, s1, s2, azeros)
    angular_aev = ang_out.reshape(N, N_PAIRS * ANG_SUB)

    return _assemble(radial_aev, angular_aev)
